# baseline (speedup 1.0000x reference)
import functools
import jax
import jax.numpy as jnp
from jax import lax
from jax.experimental import pallas as pl
from jax.experimental.pallas import tpu as pltpu, tpu_sc as plsc

_CHUNK = 128
_NBUF = 4


def kernel(tokens, embedding_weight):
    seq_len = tokens.shape[1]
    _, d_model = embedding_weight.shape
    rows_per_w = seq_len // 2
    nchunk = rows_per_w // _CHUNK
    mesh = plsc.ScalarSubcoreMesh(axis_name="c", num_cores=2)

    @functools.partial(
        pl.kernel,
        mesh=mesh,
        out_type=jax.ShapeDtypeStruct((seq_len, d_model), jnp.float32),
        scratch_types=(
            [pltpu.VMEM_SHARED((_NBUF, _CHUNK, d_model), jnp.float32)]
            + [pltpu.SemaphoreType.DMA] * (2 * _NBUF)
        ),
    )
    def k(table, out, ring, *sems):
        lsems = sems[:_NBUF]
        ssems = sems[_NBUF:]
        base = lax.axis_index("c") * rows_per_w

        def load(c):
            b = c % _NBUF
            return pltpu.make_async_copy(
                table.at[pl.ds(base + c * _CHUNK, _CHUNK)], ring.at[b], lsems[b]
            )

        def store(c):
            b = c % _NBUF
            return pltpu.make_async_copy(
                ring.at[b], out.at[pl.ds(base + c * _CHUNK, _CHUNK)], ssems[b]
            )

        for c in range(min(_NBUF, nchunk)):
            load(c).start()
        for c in range(nchunk):
            if c >= _NBUF:
                store(c - _NBUF).wait()
                load(c).start()
            load(c).wait()
            store(c).start()
        for c in range(max(nchunk - _NBUF, 0), nchunk):
            store(c).wait()

    return k(embedding_weight)[None]


# speedup vs baseline: 1.8811x; 1.8811x over previous
import jax
import jax.numpy as jnp
from jax.experimental import pallas as pl
from jax.experimental.pallas import tpu as pltpu


def _copy_block(in_ref, out_ref):
    out_ref[...] = in_ref[...]


def kernel(tokens, embedding_weight):
    seq_len = tokens.shape[1]
    _, d_model = embedding_weight.shape
    block = 1024
    out = pl.pallas_call(
        _copy_block,
        grid=(seq_len // block,),
        in_specs=[pl.BlockSpec((block, d_model), lambda i: (i, 0))],
        out_specs=pl.BlockSpec((block, d_model), lambda i: (i, 0)),
        out_shape=jax.ShapeDtypeStruct((seq_len, d_model), embedding_weight.dtype),
        compiler_params=pltpu.CompilerParams(dimension_semantics=("parallel",)),
    )(embedding_weight)
    return out[None]
